# baseline (speedup 1.0000x reference)
import functools

import jax
import jax.numpy as jnp
from jax import lax
from jax.experimental import pallas as pl
from jax.experimental.pallas import tpu as pltpu
from jax.experimental.pallas import tpu_sc as plsc

_B = 16384
_F = 32
_BIAS = 0.1
_NW = 32
_BPW = _B // _NW
_CHUNK = 128
_NCHUNK = _BPW // _CHUNK


def _mf_body(users_hbm, items_hbm, uembT_hbm, iembT_hbm, ub_hbm, ib_hbm,
             out_hbm, uidx_v, iidx_v, ubuf_v, ibuf_v, ubias_v, ibias_v,
             out_v, sem, sem_b):
  wid = lax.axis_index("s") * 2 + lax.axis_index("c")
  base = wid * _BPW

  pltpu.sync_copy(users_hbm.at[pl.ds(base, _BPW)], uidx_v)
  pltpu.sync_copy(items_hbm.at[pl.ds(base, _BPW)], iidx_v)

  bias_copies = []
  for j in range(_NCHUNK):
    s = pl.ds(j * _CHUNK, _CHUNK)
    bias_copies.append(
        pltpu.async_copy(ub_hbm.at[uidx_v.at[s]], ubias_v.at[s], sem_b))
    bias_copies.append(
        pltpu.async_copy(ib_hbm.at[iidx_v.at[s]], ibias_v.at[s], sem_b))

  copies = []
  for f in range(_F):
    src_u = uembT_hbm.at[f]
    src_i = iembT_hbm.at[f]
    for j in range(_NCHUNK):
      s = pl.ds(j * _CHUNK, _CHUNK)
      copies.append(
          pltpu.async_copy(src_u.at[uidx_v.at[s]], ubuf_v.at[f, s], sem))
      copies.append(
          pltpu.async_copy(src_i.at[iidx_v.at[s]], ibuf_v.at[f, s], sem))
  for c in bias_copies:
    c.wait()
  for c in copies:
    c.wait()

  def group(g, carry):
    js = pl.ds(g * 16, 16)
    acc = ubias_v[js] + ibias_v[js] + jnp.float32(_BIAS)
    for f in range(_F):
      acc = acc + ubuf_v[f, js] * ibuf_v[f, js]
    out_v[js] = acc
    return carry

  lax.fori_loop(0, _BPW // 16, group, 0)
  pltpu.sync_copy(out_v, out_hbm.at[pl.ds(base, _BPW)])


@jax.jit
def _mf(users, items, user_emb, item_emb, user_bias_emb, item_bias_emb):
  mesh = plsc.VectorSubcoreMesh(core_axis_name="c", subcore_axis_name="s")
  return pl.kernel(
      _mf_body,
      out_type=jax.ShapeDtypeStruct((_B,), jnp.float32),
      mesh=mesh,
      compiler_params=pltpu.CompilerParams(use_tc_tiling_on_sc=False),
      scratch_types=[
          pltpu.VMEM((_BPW,), jnp.int32),
          pltpu.VMEM((_BPW,), jnp.int32),
          pltpu.VMEM((_F, _BPW), jnp.float32),
          pltpu.VMEM((_F, _BPW), jnp.float32),
          pltpu.VMEM((_BPW,), jnp.float32),
          pltpu.VMEM((_BPW,), jnp.float32),
          pltpu.VMEM((_BPW,), jnp.float32),
          pltpu.SemaphoreType.DMA,
          pltpu.SemaphoreType.DMA,
      ],
  )(users, items, user_emb.T, item_emb.T,
    user_bias_emb.reshape(-1), item_bias_emb.reshape(-1))


def kernel(users, items, user_emb, item_emb, user_bias_emb, item_bias_emb):
  return _mf(users, items, user_emb, item_emb, user_bias_emb,
             item_bias_emb)


# speedup vs baseline: 17.6325x; 17.6325x over previous
import functools

import jax
import jax.numpy as jnp
from jax import lax
from jax.experimental import pallas as pl
from jax.experimental.pallas import tpu as pltpu
from jax.experimental.pallas import tpu_sc as plsc

_B = 16384
_F = 32
_BIAS = 0.1
_NW = 32
_BPW = _B // _NW
_CHUNK = 128
_NCHUNK = _BPW // _CHUNK
_NG = _BPW // 16

_DNUMS = lax.GatherDimensionNumbers(
    offset_dims=(), collapsed_slice_dims=(0,), start_index_map=(0,))


def _rot(x, perm):
  return lax.gather(x, perm, _DNUMS, slice_sizes=(1,),
                    mode=lax.GatherScatterMode.PROMISE_IN_BOUNDS)


def _mf_body(users_hbm, items_hbm, uembT_hbm, iembT_hbm, ub_hbm, ib_hbm,
             out_hbm, uidx_v, iidx_v, blk_v, ubias_v, ibias_v, out_v,
             s0, s1, s2, s3, s4, s5, s6, s7, sem_b):
  sems = [s0, s1, s2, s3, s4, s5, s6, s7]
  wid = lax.axis_index("s") * 2 + lax.axis_index("c")
  base = wid * _BPW

  pltpu.sync_copy(users_hbm.at[pl.ds(base, _BPW)], uidx_v.at[pl.ds(0, _BPW)])
  pltpu.sync_copy(items_hbm.at[pl.ds(base, _BPW)], iidx_v.at[pl.ds(0, _BPW)])

  bias_copies = []
  for j in range(_NCHUNK):
    s = pl.ds(j * _CHUNK, _CHUNK)
    bias_copies.append(
        pltpu.async_copy(ub_hbm.at[uidx_v.at[s]], ubias_v.at[s], sem_b))
    bias_copies.append(
        pltpu.async_copy(ib_hbm.at[iidx_v.at[s]], ibias_v.at[s], sem_b))

  def fire(u, v, us, semx):
    cbu = pl.multiple_of((u >> 7) << 7, 128)
    cbv = pl.multiple_of((v >> 7) << 7, 128)
    pltpu.async_copy(uembT_hbm.at[:, pl.ds(cbu, _CHUNK)], blk_v.at[us], semx)
    pltpu.async_copy(iembT_hbm.at[:, pl.ds(cbv, _CHUNK)], blk_v.at[us + 1],
                     semx)

  uvec0 = uidx_v[pl.ds(0, 16)]
  ivec0 = iidx_v[pl.ds(0, 16)]
  for j in range(7):
    fire(uvec0[j], ivec0[j], (2 * j) % 16, sems[j % 8])
  for c in bias_copies:
    c.wait()

  iota16 = lax.iota(jnp.int32, 16)
  dummy = uembT_hbm.at[:, pl.ds(0, _CHUNK)]

  def group(g, carry):
    gb = g * 16
    uvec_a = uidx_v[pl.ds(gb, 16)]
    ivec_a = iidx_v[pl.ds(gb, 16)]
    uvec_b = uidx_v[pl.ds(gb + 16, 16)]
    ivec_b = iidx_v[pl.ds(gb + 16, 16)]
    res = jnp.zeros((16,), jnp.float32)
    for k in range(16):
      semx = sems[k % 8]
      ucs = (2 * k) % 16
      pltpu.make_async_copy(dummy, blk_v.at[ucs], semx).wait()
      pltpu.make_async_copy(dummy, blk_v.at[ucs + 1], semx).wait()

      u = uvec_a[k]
      v = ivec_a[k]
      lu = u & 127
      lv = v & 127
      cu16 = pl.multiple_of((lu >> 4) << 4, 16)
      cv16 = pl.multiple_of((lv >> 4) << 4, 16)
      permd = ((iota16 + (lv - lu)) & 15)[:, None]
      ublk = blk_v.at[ucs]
      iblk = blk_v.at[ucs + 1]
      acc = jnp.zeros((16,), jnp.float32)
      for f in range(_F):
        cu = ublk[f, pl.ds(cu16, 16)]
        cv = iblk[f, pl.ds(cv16, 16)]
        acc = acc + cu * _rot(cv, permd)
      lanesel = ((lu & 15) + iota16 * 0)[:, None]
      tot = _rot(acc, lanesel)
      res = jnp.where(iota16 == k, tot, res)

      fs = (2 * k + 14) % 16
      fsem = sems[(k + 7) % 8]
      if k <= 8:
        fire(uvec_a[k + 7], ivec_a[k + 7], fs, fsem)
      else:
        @pl.when(g < _NG - 1)
        def _():
          fire(uvec_b[k - 9], ivec_b[k - 9], fs, fsem)

    js = pl.ds(gb, 16)
    out_v[js] = res + ubias_v[js] + ibias_v[js] + jnp.float32(_BIAS)
    return carry

  lax.fori_loop(0, _NG, group, 0)
  pltpu.sync_copy(out_v, out_hbm.at[pl.ds(base, _BPW)])


@jax.jit
def _mf(users, items, user_emb, item_emb, user_bias_emb, item_bias_emb):
  mesh = plsc.VectorSubcoreMesh(core_axis_name="c", subcore_axis_name="s")
  return pl.kernel(
      _mf_body,
      out_type=jax.ShapeDtypeStruct((_B,), jnp.float32),
      mesh=mesh,
      compiler_params=pltpu.CompilerParams(use_tc_tiling_on_sc=True),
      scratch_types=[
          pltpu.VMEM((_BPW + 32,), jnp.int32),
          pltpu.VMEM((_BPW + 32,), jnp.int32),
          pltpu.VMEM((16, _F, _CHUNK), jnp.float32),
          pltpu.VMEM((_BPW,), jnp.float32),
          pltpu.VMEM((_BPW,), jnp.float32),
          pltpu.VMEM((_BPW,), jnp.float32),
          pltpu.SemaphoreType.DMA,
          pltpu.SemaphoreType.DMA,
          pltpu.SemaphoreType.DMA,
          pltpu.SemaphoreType.DMA,
          pltpu.SemaphoreType.DMA,
          pltpu.SemaphoreType.DMA,
          pltpu.SemaphoreType.DMA,
          pltpu.SemaphoreType.DMA,
          pltpu.SemaphoreType.DMA,
      ],
  )(users, items, user_emb.T, item_emb.T,
    user_bias_emb.reshape(-1), item_bias_emb.reshape(-1))


def kernel(users, items, user_emb, item_emb, user_bias_emb, item_bias_emb):
  return _mf(users, items, user_emb, item_emb, user_bias_emb,
             item_bias_emb)


# speedup vs baseline: 17.9576x; 1.0184x over previous
import functools

import jax
import jax.numpy as jnp
from jax import lax
from jax.experimental import pallas as pl
from jax.experimental.pallas import tpu as pltpu
from jax.experimental.pallas import tpu_sc as plsc

_B = 16384
_F = 32
_BIAS = 0.1
_NW = 32
_BPW = _B // _NW
_CHUNK = 128
_NCHUNK = _BPW // _CHUNK
_NG = _BPW // 16

_DNUMS = lax.GatherDimensionNumbers(
    offset_dims=(), collapsed_slice_dims=(0,), start_index_map=(0,))


def _rot(x, perm):
  return lax.gather(x, perm, _DNUMS, slice_sizes=(1,),
                    mode=lax.GatherScatterMode.PROMISE_IN_BOUNDS)


def _mf_body(users_hbm, items_hbm, uembT_hbm, iembT_hbm, ub_hbm, ib_hbm,
             out_hbm, uidx_v, iidx_v, blk_v, ubias_v, ibias_v, out_v,
             s0, s1, s2, s3, s4, s5, s6, s7, sem_b):
  sems = [s0, s1, s2, s3, s4, s5, s6, s7]
  wid = lax.axis_index("s") * 2 + lax.axis_index("c")
  base = wid * _BPW

  pltpu.sync_copy(users_hbm.at[pl.ds(base, _BPW)], uidx_v.at[pl.ds(0, _BPW)])
  pltpu.sync_copy(items_hbm.at[pl.ds(base, _BPW)], iidx_v.at[pl.ds(0, _BPW)])

  bias_copies = []
  for j in range(_NCHUNK):
    s = pl.ds(j * _CHUNK, _CHUNK)
    bias_copies.append(
        pltpu.async_copy(ub_hbm.at[uidx_v.at[s]], ubias_v.at[s], sem_b))
    bias_copies.append(
        pltpu.async_copy(ib_hbm.at[iidx_v.at[s]], ibias_v.at[s], sem_b))

  def fire(u, v, us, semx):
    cbu = pl.multiple_of((u >> 7) << 7, 128)
    cbv = pl.multiple_of((v >> 7) << 7, 128)
    pltpu.async_copy(uembT_hbm.at[:, pl.ds(cbu, _CHUNK)], blk_v.at[us], semx)
    pltpu.async_copy(iembT_hbm.at[:, pl.ds(cbv, _CHUNK)], blk_v.at[us + 1],
                     semx)

  uvec0 = uidx_v[pl.ds(0, 16)]
  ivec0 = iidx_v[pl.ds(0, 16)]
  for j in range(7):
    fire(uvec0[j], ivec0[j], (2 * j) % 16, sems[j % 8])
  for c in bias_copies:
    c.wait()

  iota16 = lax.iota(jnp.int32, 16)
  dummy = uembT_hbm.at[:, pl.ds(0, _CHUNK)]

  def group(g, carry):
    gb = g * 16
    uvec_a = uidx_v[pl.ds(gb, 16)]
    ivec_a = iidx_v[pl.ds(gb, 16)]
    uvec_b = uidx_v[pl.ds(gb + 16, 16)]
    ivec_b = iidx_v[pl.ds(gb + 16, 16)]
    res = jnp.zeros((16,), jnp.float32)
    for k in range(16):
      fs = (2 * k + 14) % 16
      fsem = sems[(k + 7) % 8]
      if k <= 8:
        fire(uvec_a[k + 7], ivec_a[k + 7], fs, fsem)
      else:
        @pl.when(g < _NG - 1)
        def _():
          fire(uvec_b[k - 9], ivec_b[k - 9], fs, fsem)

      semx = sems[k % 8]
      ucs = (2 * k) % 16
      pltpu.make_async_copy(dummy, blk_v.at[ucs], semx).wait()
      pltpu.make_async_copy(dummy, blk_v.at[ucs + 1], semx).wait()

      u = uvec_a[k]
      v = ivec_a[k]
      lu = u & 127
      lv = v & 127
      cu16 = pl.multiple_of((lu >> 4) << 4, 16)
      cv16 = pl.multiple_of((lv >> 4) << 4, 16)
      permd = ((iota16 + (lv - lu)) & 15)[:, None]
      ublk = blk_v.at[ucs]
      iblk = blk_v.at[ucs + 1]
      acc = jnp.zeros((16,), jnp.float32)
      for f in range(_F):
        cu = ublk[f, pl.ds(cu16, 16)]
        cv = iblk[f, pl.ds(cv16, 16)]
        acc = acc + cu * _rot(cv, permd)
      lanesel = ((lu & 15) + iota16 * 0)[:, None]
      tot = _rot(acc, lanesel)
      res = jnp.where(iota16 == k, tot, res)

    js = pl.ds(gb, 16)
    out_v[js] = res + ubias_v[js] + ibias_v[js] + jnp.float32(_BIAS)
    return carry

  lax.fori_loop(0, _NG, group, 0)
  pltpu.sync_copy(out_v, out_hbm.at[pl.ds(base, _BPW)])


@jax.jit
def _mf(users, items, user_emb, item_emb, user_bias_emb, item_bias_emb):
  mesh = plsc.VectorSubcoreMesh(core_axis_name="c", subcore_axis_name="s")
  return pl.kernel(
      _mf_body,
      out_type=jax.ShapeDtypeStruct((_B,), jnp.float32),
      mesh=mesh,
      compiler_params=pltpu.CompilerParams(use_tc_tiling_on_sc=True),
      scratch_types=[
          pltpu.VMEM((_BPW + 32,), jnp.int32),
          pltpu.VMEM((_BPW + 32,), jnp.int32),
          pltpu.VMEM((16, _F, _CHUNK), jnp.float32),
          pltpu.VMEM((_BPW,), jnp.float32),
          pltpu.VMEM((_BPW,), jnp.float32),
          pltpu.VMEM((_BPW,), jnp.float32),
          pltpu.SemaphoreType.DMA,
          pltpu.SemaphoreType.DMA,
          pltpu.SemaphoreType.DMA,
          pltpu.SemaphoreType.DMA,
          pltpu.SemaphoreType.DMA,
          pltpu.SemaphoreType.DMA,
          pltpu.SemaphoreType.DMA,
          pltpu.SemaphoreType.DMA,
          pltpu.SemaphoreType.DMA,
      ],
  )(users, items, user_emb.T, item_emb.T,
    user_bias_emb.reshape(-1), item_bias_emb.reshape(-1))


def kernel(users, items, user_emb, item_emb, user_bias_emb, item_bias_emb):
  return _mf(users, items, user_emb, item_emb, user_bias_emb,
             item_bias_emb)
